# Initial kernel scaffold; baseline (speedup 1.0000x reference)
import jax
import jax.numpy as jnp
from jax.experimental import pallas as pl


def kernel(h, edge_index, edge_id, relvectors, Wq, Wk, Wv, Wa, ba, g_att, b_att, W1, b1, W2, b2, g_fin, b_fin):
    raise NotImplementedError("write your pallas kernel here")


# speedup vs baseline: 13.1332x; 13.1332x over previous
import functools
import math

import jax
import jax.numpy as jnp
from jax import lax
from jax.experimental import pallas as pl
from jax.experimental.pallas import tpu as pltpu


def _tables_body(h_ref, rel_ref, wq_ref, wk_ref, wv_ref, q_ref, kt_ref, vt_ref):
  hb = h_ref[...]
  q_ref[...] = jnp.dot(hb, wq_ref[...], preferred_element_type=jnp.float32)
  hs = hb[:, None, :] + rel_ref[...][None, :, :]
  hs = jnp.where(hs >= 0, hs, 0.25 * hs)
  hs2 = hs.reshape(-1, hs.shape[-1])
  kt_ref[...] = jnp.dot(hs2, wk_ref[...],
                        preferred_element_type=jnp.float32).reshape(hs.shape)
  vt_ref[...] = jnp.dot(hs2, wv_ref[...],
                        preferred_element_type=jnp.float32).reshape(hs.shape)


def _make_tables(h, relvectors, Wq, Wk, Wv, block_n=400):
  N, H = h.shape
  NR = relvectors.shape[0]
  grid = N // block_n
  q, kt, vt = pl.pallas_call(
      _tables_body,
      grid=(grid,),
      in_specs=[
          pl.BlockSpec((block_n, H), lambda i: (i, 0)),
          pl.BlockSpec((NR, H), lambda i: (0, 0)),
          pl.BlockSpec((H, H), lambda i: (0, 0)),
          pl.BlockSpec((H, H), lambda i: (0, 0)),
          pl.BlockSpec((H, H), lambda i: (0, 0)),
      ],
      out_specs=[
          pl.BlockSpec((block_n, H), lambda i: (i, 0)),
          pl.BlockSpec((block_n, NR, H), lambda i: (i, 0, 0)),
          pl.BlockSpec((block_n, NR, H), lambda i: (i, 0, 0)),
      ],
      out_shape=[
          jax.ShapeDtypeStruct((N, H), jnp.float32),
          jax.ShapeDtypeStruct((N, NR, H), jnp.float32),
          jax.ShapeDtypeStruct((N, NR, H), jnp.float32),
      ],
  )(h, relvectors, Wq, Wk, Wv)
  return q, kt, vt


def _sr_body(src_ref, rid_ref, sr_ref, *, nr):
  sr_ref[...] = src_ref[...] * nr + rid_ref[...]


def _make_sr(src, rid, nr):
  E = src.shape[0]
  src2 = src.reshape(E // 128, 128)
  rid2 = rid.reshape(E // 128, 128)
  sr = pl.pallas_call(
      functools.partial(_sr_body, nr=nr),
      out_shape=jax.ShapeDtypeStruct((E // 128, 128), jnp.int32),
  )(src2, rid2)
  return sr.reshape(E)


def _ln(x, g, b, eps=1e-5):
  mu = jnp.mean(x, axis=-1, keepdims=True)
  var = jnp.mean((x - mu) ** 2, axis=-1, keepdims=True)
  return (x - mu) * jax.lax.rsqrt(var + eps) * g + b


def _final_body(red_ref, den_ref, h_ref, wa_ref, ba_ref, ga_ref, bba_ref,
                w1_ref, b1_ref, w2_ref, b2_ref, gf_ref, bf_ref, out_ref):
  den = den_ref[...]
  B = den.shape[0]
  nh = 4
  dh = 32
  denb = jnp.broadcast_to(den[:, :nh, None], (B, nh, dh)).reshape(B, nh * dh)
  red = red_ref[...] / (denb + 1e-20)
  summ = jnp.dot(red, wa_ref[...], preferred_element_type=jnp.float32) \
      + ba_ref[...]
  hh = _ln(summ + h_ref[...], ga_ref[...], bba_ref[...])
  x = jnp.dot(hh, w1_ref[...], preferred_element_type=jnp.float32) + b1_ref[...]
  x = jnp.where(x > 0, x, jnp.exp(jnp.minimum(x, 0.0)) - 1.0)
  x = jnp.dot(x, w2_ref[...], preferred_element_type=jnp.float32) + b2_ref[...]
  out_ref[...] = _ln(hh + x, gf_ref[...], bf_ref[...])


def _final_phase(red, den8, h, Wa, ba, g_att, b_att, W1, b1, W2, b2,
                 g_fin, b_fin, block_n=400):
  N, H = h.shape
  Z = W1.shape[1]
  grid = N // block_n
  row = lambda v: v.reshape(1, -1)
  out = pl.pallas_call(
      _final_body,
      grid=(grid,),
      in_specs=[
          pl.BlockSpec((block_n, H), lambda i: (i, 0)),
          pl.BlockSpec((block_n, 8), lambda i: (i, 0)),
          pl.BlockSpec((block_n, H), lambda i: (i, 0)),
          pl.BlockSpec((H, H), lambda i: (0, 0)),
          pl.BlockSpec((1, H), lambda i: (0, 0)),
          pl.BlockSpec((1, H), lambda i: (0, 0)),
          pl.BlockSpec((1, H), lambda i: (0, 0)),
          pl.BlockSpec((H, Z), lambda i: (0, 0)),
          pl.BlockSpec((1, Z), lambda i: (0, 0)),
          pl.BlockSpec((Z, H), lambda i: (0, 0)),
          pl.BlockSpec((1, H), lambda i: (0, 0)),
          pl.BlockSpec((1, H), lambda i: (0, 0)),
          pl.BlockSpec((1, H), lambda i: (0, 0)),
      ],
      out_specs=pl.BlockSpec((block_n, H), lambda i: (i, 0)),
      out_shape=jax.ShapeDtypeStruct((N, H), jnp.float32),
  )(red, den8, h, Wa, row(ba), row(g_att), row(b_att), W1, row(b1),
    W2, row(b2), row(g_fin), row(b_fin))
  return out


def _edge_phase(q, ktab, vtab, sr, dst, nh, dh):
  E = sr.shape[0]
  N, H = q.shape
  kr = ktab.reshape(-1, H)[sr].reshape(E, nh, dh)
  qr = q[dst].reshape(E, nh, dh)
  w = (kr * qr).sum(-1) / math.sqrt(dh)
  ew = jnp.exp(w)
  den = jax.ops.segment_sum(ew, dst, num_segments=N)
  vr = vtab.reshape(-1, H)[sr].reshape(E, nh, dh)
  red = jax.ops.segment_sum((ew[..., None] * vr).reshape(E, H), dst,
                            num_segments=N)
  den8 = jnp.pad(den, ((0, 0), (0, 8 - nh)))
  return red, den8


def kernel(h, edge_index, edge_id, relvectors, Wq, Wk, Wv, Wa, ba, g_att,
           b_att, W1, b1, W2, b2, g_fin, b_fin):
  N, H = h.shape
  NR = relvectors.shape[0]
  nh = 4
  dh = H // nh
  src = edge_index[0]
  dst = edge_index[1]

  q, ktab, vtab = _make_tables(h, relvectors, Wq, Wk, Wv)
  sr = _make_sr(src, edge_id, NR)
  red, den8 = _edge_phase(q, ktab, vtab, sr, dst, nh, dh)
  return _final_phase(red, den8, h, Wa, ba, g_att, b_att, W1, b1, W2, b2,
                      g_fin, b_fin)


# speedup vs baseline: 14.8952x; 1.1342x over previous
import functools
import math

import jax
import jax.numpy as jnp
from jax import lax
from jax.experimental import pallas as pl
from jax.experimental.pallas import tpu as pltpu
from jax.experimental.pallas import tpu_sc as plsc

_NC = 2
_NS = 16
_NW = _NC * _NS


def _make_gather3(E, R, H, C):
  per_w = E // _NW
  n_chunks = per_w // C
  mesh = plsc.VectorSubcoreMesh(core_axis_name="c", subcore_axis_name="s")
  f32 = jnp.float32

  @functools.partial(
      pl.kernel, mesh=mesh,
      out_type=[
          jax.ShapeDtypeStruct((E, H), f32),
          jax.ShapeDtypeStruct((E, H), f32),
          jax.ShapeDtypeStruct((E, H), f32),
      ],
      scratch_types=[
          pltpu.VMEM((C,), jnp.int32),
          pltpu.VMEM((C,), jnp.int32),
          pltpu.VMEM((C, H), f32),
          pltpu.VMEM((C, H), f32),
          pltpu.VMEM((C, H), f32),
          pltpu.SemaphoreType.DMA,
      ],
  )
  def gather3(ktab_hbm, q_hbm, vtab_hbm, sr_hbm, dst_hbm,
              kout, qout, vout, sr_v, dst_v, krows, qrows, vrows, sem):
    wid = lax.axis_index("s") * _NC + lax.axis_index("c")
    base_w = wid * per_w

    def body(i, _):
      base = base_w + i * C
      pltpu.sync_copy(sr_hbm.at[pl.ds(base, C)], sr_v)
      pltpu.sync_copy(dst_hbm.at[pl.ds(base, C)], dst_v)
      ck = pltpu.async_copy(ktab_hbm.at[sr_v], krows, sem)
      cq = pltpu.async_copy(q_hbm.at[dst_v], qrows, sem)
      cv = pltpu.async_copy(vtab_hbm.at[sr_v], vrows, sem)
      ck.wait()
      cq.wait()
      cv.wait()
      pltpu.sync_copy(krows, kout.at[pl.ds(base, C)])
      pltpu.sync_copy(qrows, qout.at[pl.ds(base, C)])
      pltpu.sync_copy(vrows, vout.at[pl.ds(base, C)])
      return _

    lax.fori_loop(0, n_chunks, body, None)

  return gather3


def _tables_body(h_ref, rel_ref, wq_ref, wk_ref, wv_ref, q_ref, kt_ref, vt_ref):
  hb = h_ref[...]
  q_ref[...] = jnp.dot(hb, wq_ref[...],
                       preferred_element_type=jnp.float32) * (1.0 / math.sqrt(32.0))
  hs = hb[:, None, :] + rel_ref[...][None, :, :]
  hs = jnp.where(hs >= 0, hs, 0.25 * hs)
  hs2 = hs.reshape(-1, hs.shape[-1])
  kt_ref[...] = jnp.dot(hs2, wk_ref[...],
                        preferred_element_type=jnp.float32).reshape(hs.shape)
  vt_ref[...] = jnp.dot(hs2, wv_ref[...],
                        preferred_element_type=jnp.float32).reshape(hs.shape)


def _make_tables(h, relvectors, Wq, Wk, Wv, block_n=400):
  N, H = h.shape
  NR = relvectors.shape[0]
  grid = N // block_n
  q, kt, vt = pl.pallas_call(
      _tables_body,
      grid=(grid,),
      in_specs=[
          pl.BlockSpec((block_n, H), lambda i: (i, 0)),
          pl.BlockSpec((NR, H), lambda i: (0, 0)),
          pl.BlockSpec((H, H), lambda i: (0, 0)),
          pl.BlockSpec((H, H), lambda i: (0, 0)),
          pl.BlockSpec((H, H), lambda i: (0, 0)),
      ],
      out_specs=[
          pl.BlockSpec((block_n, H), lambda i: (i, 0)),
          pl.BlockSpec((block_n, NR, H), lambda i: (i, 0, 0)),
          pl.BlockSpec((block_n, NR, H), lambda i: (i, 0, 0)),
      ],
      out_shape=[
          jax.ShapeDtypeStruct((N, H), jnp.float32),
          jax.ShapeDtypeStruct((N, NR, H), jnp.float32),
          jax.ShapeDtypeStruct((N, NR, H), jnp.float32),
      ],
  )(h, relvectors, Wq, Wk, Wv)
  return q, kt, vt


def _ew_scale_body(k_ref, q_ref, v_ref, ew_ref, sc_ref):
  kq = k_ref[...] * q_ref[...]
  B = kq.shape[0]
  w = jnp.sum(kq.reshape(B, 4, 32), axis=-1)
  ew = jnp.exp(w)
  ew_ref[...] = ew
  ewb = jnp.broadcast_to(ew[:, :, None], (B, 4, 32)).reshape(B, 128)
  sc_ref[...] = v_ref[...] * ewb


def _ew_scale(krows, qrows, vrows, block_e=2000):
  E, H = krows.shape
  grid = E // block_e
  ew4, scaled = pl.pallas_call(
      _ew_scale_body,
      grid=(grid,),
      in_specs=[
          pl.BlockSpec((block_e, H), lambda i: (i, 0)),
          pl.BlockSpec((block_e, H), lambda i: (i, 0)),
          pl.BlockSpec((block_e, H), lambda i: (i, 0)),
      ],
      out_specs=[
          pl.BlockSpec((block_e, 4), lambda i: (i, 0)),
          pl.BlockSpec((block_e, H), lambda i: (i, 0)),
      ],
      out_shape=[
          jax.ShapeDtypeStruct((E, 4), jnp.float32),
          jax.ShapeDtypeStruct((E, H), jnp.float32),
      ],
  )(krows, qrows, vrows)
  return ew4, scaled


def _sr_body(src_ref, rid_ref, sr_ref, *, nr):
  sr_ref[...] = src_ref[...] * nr + rid_ref[...]


def _make_sr(src, rid, nr):
  E = src.shape[0]
  src2 = src.reshape(E // 128, 128)
  rid2 = rid.reshape(E // 128, 128)
  sr = pl.pallas_call(
      functools.partial(_sr_body, nr=nr),
      out_shape=jax.ShapeDtypeStruct((E // 128, 128), jnp.int32),
  )(src2, rid2)
  return sr.reshape(E)


def _ln(x, g, b, eps=1e-5):
  mu = jnp.mean(x, axis=-1, keepdims=True)
  var = jnp.mean((x - mu) ** 2, axis=-1, keepdims=True)
  return (x - mu) * jax.lax.rsqrt(var + eps) * g + b


def _final_body(red_ref, den_ref, h_ref, wa_ref, ba_ref, ga_ref, bba_ref,
                w1_ref, b1_ref, w2_ref, b2_ref, gf_ref, bf_ref, out_ref):
  den = den_ref[...]
  B = den.shape[0]
  nh = 4
  dh = 32
  denb = jnp.broadcast_to(den[:, :, None], (B, nh, dh)).reshape(B, nh * dh)
  red = red_ref[...] / (denb + 1e-20)
  summ = jnp.dot(red, wa_ref[...], preferred_element_type=jnp.float32) \
      + ba_ref[...]
  hh = _ln(summ + h_ref[...], ga_ref[...], bba_ref[...])
  x = jnp.dot(hh, w1_ref[...], preferred_element_type=jnp.float32) + b1_ref[...]
  x = jnp.where(x > 0, x, jnp.exp(jnp.minimum(x, 0.0)) - 1.0)
  x = jnp.dot(x, w2_ref[...], preferred_element_type=jnp.float32) + b2_ref[...]
  out_ref[...] = _ln(hh + x, gf_ref[...], bf_ref[...])


def _final_phase(red, den, h, Wa, ba, g_att, b_att, W1, b1, W2, b2,
                 g_fin, b_fin, block_n=400):
  N, H = h.shape
  Z = W1.shape[1]
  grid = N // block_n
  row = lambda v: v.reshape(1, -1)
  out = pl.pallas_call(
      _final_body,
      grid=(grid,),
      in_specs=[
          pl.BlockSpec((block_n, H), lambda i: (i, 0)),
          pl.BlockSpec((block_n, 4), lambda i: (i, 0)),
          pl.BlockSpec((block_n, H), lambda i: (i, 0)),
          pl.BlockSpec((H, H), lambda i: (0, 0)),
          pl.BlockSpec((1, H), lambda i: (0, 0)),
          pl.BlockSpec((1, H), lambda i: (0, 0)),
          pl.BlockSpec((1, H), lambda i: (0, 0)),
          pl.BlockSpec((H, Z), lambda i: (0, 0)),
          pl.BlockSpec((1, Z), lambda i: (0, 0)),
          pl.BlockSpec((Z, H), lambda i: (0, 0)),
          pl.BlockSpec((1, H), lambda i: (0, 0)),
          pl.BlockSpec((1, H), lambda i: (0, 0)),
          pl.BlockSpec((1, H), lambda i: (0, 0)),
      ],
      out_specs=pl.BlockSpec((block_n, H), lambda i: (i, 0)),
      out_shape=jax.ShapeDtypeStruct((N, H), jnp.float32),
  )(red, den, h, Wa, row(ba), row(g_att), row(b_att), W1, row(b1),
    W2, row(b2), row(g_fin), row(b_fin))
  return out


def kernel(h, edge_index, edge_id, relvectors, Wq, Wk, Wv, Wa, ba, g_att,
           b_att, W1, b1, W2, b2, g_fin, b_fin):
  N, H = h.shape
  NR = relvectors.shape[0]
  src = edge_index[0]
  dst = edge_index[1]

  q, ktab, vtab = _make_tables(h, relvectors, Wq, Wk, Wv)
  sr = _make_sr(src, edge_id, NR)
  gather3 = _make_gather3(src.shape[0], N * NR, H, C=80)
  krows, qrows, vrows = gather3(ktab.reshape(-1, H), q, vtab.reshape(-1, H),
                                sr, dst)
  ew4, scaled = _ew_scale(krows, qrows, vrows)
  red = jax.ops.segment_sum(scaled, dst, num_segments=N)
  den = jax.ops.segment_sum(ew4, dst, num_segments=N)
  return _final_phase(red, den, h, Wa, ba, g_att, b_att, W1, b1, W2, b2,
                      g_fin, b_fin)
